# baseline (speedup 1.0000x reference)
import functools

import jax
import jax.numpy as jnp
from jax import lax
from jax.experimental import pallas as pl
from jax.experimental.pallas import tpu as pltpu
from jax.experimental.pallas import tpu_sc as plsc

R = 32
RP = R + 1
NB_BLK = 400
Z_BLK = 1000
NW = 32
CH = 80
CHM = 128
EPT = 10240


def _mesh():
    return plsc.VectorSubcoreMesh(core_axis_name="c", subcore_axis_name="s")


def _hist_body(N, E, dst_hbm, et_hbm, cnt_hbm, dpre, tpre, kbuf2, ones, zbuf, cnt_sp):
    cid = lax.axis_index("c")
    sid = lax.axis_index("s")
    wid = cid * 16 + sid
    NR = N * R
    per_tile = NR // 16
    seg = per_tile // 10
    epw = E // NW
    nch = epw // CH

    for i in range(CH // 16):
        ones[pl.ds(i * 16, 16)] = jnp.ones((16,), jnp.float32)

    def zb(i, _):
        zbuf[pl.ds(i * 16, 16)] = jnp.zeros((16,), jnp.float32)
        return 0
    lax.fori_loop(0, seg // 16, zb, 0)

    def zcopy(k, _):
        pltpu.sync_copy(zbuf, cnt_sp.at[pl.ds(sid * per_tile + k * seg, seg)])
        return 0
    lax.fori_loop(0, 10, zcopy, 0)

    pltpu.sync_copy(dst_hbm.at[pl.ds(wid * epw, epw)], dpre)
    pltpu.sync_copy(et_hbm.at[pl.ds(wid * epw, epw)], tpre)

    def keys(j, _):
        for g in range(CH // 16):
            fl = pl.ds(j * CH + g * 16, 16)
            kbuf2[j, pl.ds(g * 16, 16)] = dpre[fl] * R + tpre[fl]
        return 0
    lax.fori_loop(0, nch, keys, 0)
    plsc.subcore_barrier()

    def run(sem):
        def fire(j, _):
            pltpu.async_copy(ones, cnt_sp.at[kbuf2.at[j]], sem, add=True)
            return 0
        lax.fori_loop(0, nch, fire, 0)
        pltpu.make_async_copy(dst_hbm.at[pl.ds(0, epw)], dpre, sem).wait()
    pl.run_scoped(run, pltpu.SemaphoreType.DMA)
    plsc.subcore_barrier()

    def ecopy(k, _):
        off = sid * per_tile + k * seg
        pltpu.sync_copy(cnt_sp.at[pl.ds(off, seg)], zbuf)
        pltpu.sync_copy(zbuf, cnt_hbm.at[pl.ds(cid * NR + off, seg)])
        return 0
    lax.fori_loop(0, 10, ecopy, 0)


def _make_hist(N, E):
    NR = N * R
    seg = NR // 160
    epw = E // NW
    return pl.kernel(
        functools.partial(_hist_body, N, E),
        out_type=jax.ShapeDtypeStruct((2 * NR,), jnp.float32),
        mesh=_mesh(),
        scratch_types=[
            pltpu.VMEM((epw,), jnp.int32),
            pltpu.VMEM((epw,), jnp.int32),
            pltpu.VMEM((epw // CH, CH), jnp.int32),
            pltpu.VMEM((CH,), jnp.float32),
            pltpu.VMEM((seg,), jnp.float32),
            pltpu.VMEM_SHARED((NR,), jnp.float32),
        ],
    )


def _scale_body(N, E, src_hbm, dst_hbm, et_hbm, cnt_hbm, scale_hbm, gkey_hbm,
                dstp_hbm, c0, c1, spre, dpre, tpre, kbuf2, gflat, dflat, scflat,
                inv_sp):
    cid = lax.axis_index("c")
    sid = lax.axis_index("s")
    wid = cid * 16 + sid
    NR = N * R
    per_tile = NR // 16
    seg = per_tile // 10
    epw = E // NW
    nch = epw // CH

    def inv_seg(k, _):
        off = sid * per_tile + k * seg
        sl = pl.ds(off, seg)
        pltpu.sync_copy(cnt_hbm.at[sl], c0)
        pltpu.sync_copy(cnt_hbm.at[pl.ds(NR + off, seg)], c1)

        def body(i, _):
            s16 = pl.ds(i * 16, 16)
            tot = c0[s16] + c1[s16]
            c0[s16] = 1.0 / jnp.maximum(tot, 1.0)
            return 0
        lax.fori_loop(0, seg // 16, body, 0)
        pltpu.sync_copy(c0, inv_sp.at[sl])
        return 0
    lax.fori_loop(0, 10, inv_seg, 0)
    plsc.subcore_barrier()

    pltpu.sync_copy(src_hbm.at[pl.ds(wid * epw, epw)], spre)
    pltpu.sync_copy(dst_hbm.at[pl.ds(wid * epw, epw)], dpre)
    pltpu.sync_copy(et_hbm.at[pl.ds(wid * epw, epw)], tpre)

    def keys(j, _):
        for g in range(CH // 16):
            fl = pl.ds(j * CH + g * 16, 16)
            kbuf2[j, pl.ds(g * 16, 16)] = dpre[fl] * R + tpre[fl]
            gflat[fl] = tpre[fl] + spre[fl] * RP
            dflat[fl] = dpre[fl]
        return 0
    lax.fori_loop(0, nch, keys, 0)
    for i in range((EPT - 10000) // 16):
        fl = pl.ds(10000 + i * 16, 16)
        gflat[fl] = jnp.zeros((16,), jnp.int32)
        dflat[fl] = jnp.zeros((16,), jnp.int32)
        scflat[fl] = jnp.zeros((16,), jnp.float32)

    def run(sem):
        def fire(j, _):
            pltpu.async_copy(inv_sp.at[kbuf2.at[j]],
                             scflat.at[pl.ds(j * CH, CH)], sem)
            return 0
        lax.fori_loop(0, nch, fire, 0)
        pltpu.make_async_copy(scale_hbm.at[pl.ds(0, epw)],
                              scflat.at[pl.ds(0, epw)], sem).wait()
    pl.run_scoped(run, pltpu.SemaphoreType.DMA)

    base = wid * EPT
    pltpu.sync_copy(scflat, scale_hbm.at[pl.ds(base, EPT)])
    pltpu.sync_copy(gflat, gkey_hbm.at[pl.ds(base, EPT)])
    pltpu.sync_copy(dflat, dstp_hbm.at[pl.ds(base, EPT)])


def _make_scale(N, E):
    NR = N * R
    seg = NR // 160
    epw = E // NW
    E2 = NW * EPT
    return pl.kernel(
        functools.partial(_scale_body, N, E),
        out_type=(jax.ShapeDtypeStruct((E2,), jnp.float32),
                  jax.ShapeDtypeStruct((E2,), jnp.int32),
                  jax.ShapeDtypeStruct((E2,), jnp.int32)),
        mesh=_mesh(),
        scratch_types=[
            pltpu.VMEM((seg,), jnp.float32),
            pltpu.VMEM((seg,), jnp.float32),
            pltpu.VMEM((epw,), jnp.int32),
            pltpu.VMEM((epw,), jnp.int32),
            pltpu.VMEM((epw,), jnp.int32),
            pltpu.VMEM((epw // CH, CH), jnp.int32),
            pltpu.VMEM((EPT,), jnp.int32),
            pltpu.VMEM((EPT,), jnp.int32),
            pltpu.VMEM((EPT,), jnp.float32),
            pltpu.VMEM_SHARED((NR,), jnp.float32),
        ],
    )


PASSES = 10


def _agg_body(N, D, zt_hbm, gkey_hbm, dstp_hbm, scale_hbm, part_hbm,
              gk2, dst2, sc2, rowsA, rowsB, acc_sp):
    cid = lax.axis_index("c")
    sid = lax.axis_index("s")
    wid = cid * 16 + sid
    nch = EPT // CHM
    cpp = nch // PASSES
    seg_rows = 80
    row_base = sid * 640
    nseg = jnp.where(sid < 15, 8, (N - 15 * 640) // seg_rows)

    def zb(i, _):
        for f in range(D // 16):
            rowsA[i, pl.ds(f * 16, 16)] = jnp.zeros((16,), jnp.float32)
        return 0
    lax.fori_loop(0, seg_rows, zb, 0)

    def zcopy(k, _):
        pltpu.sync_copy(rowsA.at[pl.ds(0, seg_rows)],
                        acc_sp.at[pl.ds(row_base + k * seg_rows, seg_rows)])
        return 0
    lax.fori_loop(0, nseg, zcopy, 0)
    plsc.subcore_barrier()

    def scale_rows(rows, j):
        def grp(g, _):
            svec = sc2[j, pl.ds(g * 16, 16)]
            for i16 in range(16):
                s = jnp.take_along_axis(
                    svec, jnp.full((16,), i16, jnp.int32), axis=0)
                row = g * 16 + i16
                for f in range(D // 16):
                    sl = pl.ds(f * 16, 16)
                    rows[row, sl] = rows[row, sl] * s
            return 0
        lax.fori_loop(0, CHM // 16, grp, 0)

    SS = 4
    sr = CHM // SS

    def run(gsA, gsB, ssA, ssB):
        def gather(rows, j, sem):
            for q in range(SS):
                pltpu.async_copy(zt_hbm.at[gk2.at[j * SS + q]],
                                 rows.at[pl.ds(q * sr, sr)], sem)

        def gwait(rows, j, sem):
            pltpu.make_async_copy(zt_hbm.at[gk2.at[j * SS]], rows, sem).wait()

        def scat(rows, j, sem):
            for q in range(SS):
                pltpu.async_copy(rows.at[pl.ds(q * sr, sr)],
                                 acc_sp.at[dst2.at[j * SS + q]], sem, add=True)

        def swait(rows, j, sem):
            pltpu.make_async_copy(rows, acc_sp.at[dst2.at[j * SS]], sem).wait()

        for p in range(PASSES):
            rb = wid * nch + p * cpp
            pltpu.sync_copy(gkey_hbm.at[pl.ds(rb * SS, cpp * SS)], gk2)
            pltpu.sync_copy(dstp_hbm.at[pl.ds(rb * SS, cpp * SS)], dst2)
            pltpu.sync_copy(scale_hbm.at[pl.ds(rb, cpp)], sc2)
            gather(rowsA, 0, gsA)

            def pair(jj, _):
                j0 = jj * 2
                j1 = j0 + 1

                @pl.when(jj > 0)
                def _():
                    swait(rowsB, j0 - 1, ssB)
                gather(rowsB, j1, gsB)
                gwait(rowsA, j0, gsA)
                scale_rows(rowsA, j0)
                scat(rowsA, j0, ssA)
                gwait(rowsB, j1, gsB)
                scale_rows(rowsB, j1)
                scat(rowsB, j1, ssB)
                swait(rowsA, j0, ssA)

                @pl.when(jj < cpp // 2 - 1)
                def _():
                    gather(rowsA, j0 + 2, gsA)
                return 0
            lax.fori_loop(0, cpp // 2, pair, 0)
            swait(rowsB, cpp - 1, ssB)

    pl.run_scoped(run, pltpu.SemaphoreType.DMA, pltpu.SemaphoreType.DMA,
                  pltpu.SemaphoreType.DMA, pltpu.SemaphoreType.DMA)
    plsc.subcore_barrier()

    def ecopy(k, _):
        sl = pl.ds(row_base + k * seg_rows, seg_rows)
        pltpu.sync_copy(acc_sp.at[sl], rowsA.at[pl.ds(0, seg_rows)])
        pltpu.sync_copy(rowsA.at[pl.ds(0, seg_rows)], part_hbm.at[cid, sl])
        return 0
    lax.fori_loop(0, nseg, ecopy, 0)


def _make_agg(N, D):
    nch = EPT // CHM
    cpp = nch // PASSES
    return pl.kernel(
        functools.partial(_agg_body, N, D),
        out_type=jax.ShapeDtypeStruct((2, N, D), jnp.float32),
        mesh=_mesh(),
        scratch_types=[
            pltpu.VMEM((cpp * 4, CHM // 4), jnp.int32),
            pltpu.VMEM((cpp * 4, CHM // 4), jnp.int32),
            pltpu.VMEM((cpp, CHM), jnp.float32),
            pltpu.VMEM((CHM, D), jnp.float32),
            pltpu.VMEM((CHM, D), jnp.float32),
            pltpu.VMEM_SHARED((N, D), jnp.float32),
        ],
    )


def _wcomb_kernel(comp_ref, bases_ref, root_ref, out_ref):
    B, I, O = bases_ref.shape
    bflat = bases_ref[...].reshape(B, I * O)
    w = jnp.dot(comp_ref[...], bflat, preferred_element_type=jnp.float32)
    out_ref[0:R] = w.reshape(R, I, O)
    out_ref[R] = root_ref[...]


def _wcomb(comp, bases, root):
    B, I, O = bases.shape
    return pl.pallas_call(
        _wcomb_kernel,
        out_shape=jax.ShapeDtypeStruct((RP, I, O), jnp.float32),
    )(comp, bases, root)


def _z_kernel(f_ref, w_ref, out_ref):
    f = f_ref[...]
    for r in range(RP):
        out_ref[:, r, :] = jnp.dot(f, w_ref[r], preferred_element_type=jnp.float32)


def _z_all(feats, wfull):
    N, I = feats.shape
    O = wfull.shape[2]
    nb = N // Z_BLK
    return pl.pallas_call(
        _z_kernel,
        grid=(nb,),
        in_specs=[
            pl.BlockSpec((Z_BLK, I), lambda n: (n, 0)),
            pl.BlockSpec((RP, I, O), lambda n: (0, 0, 0)),
        ],
        out_specs=pl.BlockSpec((Z_BLK, RP, O), lambda n: (n, 0, 0)),
        out_shape=jax.ShapeDtypeStruct((N, RP, O), jnp.float32),
    )(feats, wfull)


def _comb_kernel(part_ref, z_ref, b_ref, out_ref):
    s = part_ref[0] + part_ref[1] + z_ref[...] + b_ref[...]
    out_ref[...] = jnp.maximum(s, 0.0)


def _combine(part, zroot, bias):
    _, N, O = part.shape
    nb = N // NB_BLK
    return pl.pallas_call(
        _comb_kernel,
        grid=(nb,),
        in_specs=[
            pl.BlockSpec((2, NB_BLK, O), lambda n: (0, n, 0)),
            pl.BlockSpec((NB_BLK, O), lambda n: (n, 0)),
            pl.BlockSpec((1, O), lambda n: (0, 0)),
        ],
        out_specs=pl.BlockSpec((NB_BLK, O), lambda n: (n, 0)),
        out_shape=jax.ShapeDtypeStruct((N, O), jnp.float32),
    )(part, zroot, bias.reshape(1, O))


def _final_kernel(part_ref, z_ref, b_ref, wc_ref, bc_ref, out_ref):
    h = part_ref[0] + part_ref[1] + z_ref[...] + b_ref[...]
    out_ref[...] = jnp.dot(h, wc_ref[...], preferred_element_type=jnp.float32) + bc_ref[...]


def _final(part, zroot, bias, Wc, bc):
    _, N, O = part.shape
    A = Wc.shape[1]
    nb = N // NB_BLK
    return pl.pallas_call(
        _final_kernel,
        grid=(nb,),
        in_specs=[
            pl.BlockSpec((2, NB_BLK, O), lambda n: (0, n, 0)),
            pl.BlockSpec((NB_BLK, O), lambda n: (n, 0)),
            pl.BlockSpec((1, O), lambda n: (0, 0)),
            pl.BlockSpec((O, A), lambda n: (0, 0)),
            pl.BlockSpec((1, A), lambda n: (0, 0)),
        ],
        out_specs=pl.BlockSpec((NB_BLK, A), lambda n: (n, 0)),
        out_shape=jax.ShapeDtypeStruct((N, A), jnp.float32),
    )(part, zroot, bias.reshape(1, O), Wc, bc.reshape(1, A))


@jax.jit
def kernel(x, edge_index, edge_type, bases1, comp1, root1, bias1,
           bases2, comp2, root2, bias2, Wc, bc):
    N, I = x.shape
    E = edge_index.shape[1]
    nch = EPT // CHM
    src = edge_index[0]
    dst = edge_index[1]
    et = edge_type

    cnt = _make_hist(N, E)(dst, et)
    scale, gkey, dstp = _make_scale(N, E)(src, dst, et, cnt)
    gkey2 = gkey.reshape(NW * nch * 4, CHM // 4)
    dstp2 = dstp.reshape(NW * nch * 4, CHM // 4)
    scale2 = scale.reshape(NW * nch, CHM)

    agg = _make_agg(N, I)

    w1 = _wcomb(comp1, bases1, root1)
    z1 = _z_all(x, w1)
    part1 = agg(z1.reshape(N * RP, I), gkey2, dstp2, scale2)
    h = _combine(part1, z1[:, R, :], bias1)

    w2 = _wcomb(comp2, bases2, root2)
    z2 = _z_all(h, w2)
    part2 = agg(z2.reshape(N * RP, I), gkey2, dstp2, scale2)
    return _final(part2, z2[:, R, :], bias2, Wc, bc)


# speedup vs baseline: 1.3135x; 1.3135x over previous
import functools

import jax
import jax.numpy as jnp
from jax import lax
from jax.experimental import pallas as pl
from jax.experimental.pallas import tpu as pltpu
from jax.experimental.pallas import tpu_sc as plsc

R = 32
RP = R + 1
NB_BLK = 400
Z_BLK = 1000
NW = 32
CH = 80
CHM = 128
EPT = 10240


def _mesh():
    return plsc.VectorSubcoreMesh(core_axis_name="c", subcore_axis_name="s")


def _hist_body(N, E, dst_hbm, et_hbm, cnt_hbm, dpre, tpre, kbuf2, ones, zbuf, cnt_sp):
    cid = lax.axis_index("c")
    sid = lax.axis_index("s")
    wid = cid * 16 + sid
    NR = N * R
    per_tile = NR // 16
    seg = per_tile // 10
    epw = E // NW
    nch = epw // CH

    for i in range(CH // 16):
        ones[pl.ds(i * 16, 16)] = jnp.ones((16,), jnp.float32)

    def zb(i, _):
        zbuf[pl.ds(i * 16, 16)] = jnp.zeros((16,), jnp.float32)
        return 0
    lax.fori_loop(0, seg // 16, zb, 0)

    def zcopy(k, _):
        pltpu.sync_copy(zbuf, cnt_sp.at[pl.ds(sid * per_tile + k * seg, seg)])
        return 0
    lax.fori_loop(0, 10, zcopy, 0)

    pltpu.sync_copy(dst_hbm.at[pl.ds(wid * epw, epw)], dpre)
    pltpu.sync_copy(et_hbm.at[pl.ds(wid * epw, epw)], tpre)

    def keys(j, _):
        for g in range(CH // 16):
            fl = pl.ds(j * CH + g * 16, 16)
            kbuf2[j, pl.ds(g * 16, 16)] = dpre[fl] * R + tpre[fl]
        return 0
    lax.fori_loop(0, nch, keys, 0)
    plsc.subcore_barrier()

    def run(sem):
        def fire(j, _):
            pltpu.async_copy(ones, cnt_sp.at[kbuf2.at[j]], sem, add=True)
            return 0
        lax.fori_loop(0, nch, fire, 0)
        pltpu.make_async_copy(dst_hbm.at[pl.ds(0, epw)], dpre, sem).wait()
    pl.run_scoped(run, pltpu.SemaphoreType.DMA)
    plsc.subcore_barrier()

    def ecopy(k, _):
        off = sid * per_tile + k * seg
        pltpu.sync_copy(cnt_sp.at[pl.ds(off, seg)], zbuf)
        pltpu.sync_copy(zbuf, cnt_hbm.at[pl.ds(cid * NR + off, seg)])
        return 0
    lax.fori_loop(0, 10, ecopy, 0)


def _make_hist(N, E):
    NR = N * R
    seg = NR // 160
    epw = E // NW
    return pl.kernel(
        functools.partial(_hist_body, N, E),
        out_type=jax.ShapeDtypeStruct((2 * NR,), jnp.float32),
        mesh=_mesh(),
        scratch_types=[
            pltpu.VMEM((epw,), jnp.int32),
            pltpu.VMEM((epw,), jnp.int32),
            pltpu.VMEM((epw // CH, CH), jnp.int32),
            pltpu.VMEM((CH,), jnp.float32),
            pltpu.VMEM((seg,), jnp.float32),
            pltpu.VMEM_SHARED((NR,), jnp.float32),
        ],
    )


def _scale_body(N, E, src_hbm, dst_hbm, et_hbm, cnt_hbm, scale_hbm, gkey_hbm,
                dstp_hbm, c0, c1, spre, dpre, tpre, kbuf2, gflat, dflat, scflat,
                inv_sp):
    cid = lax.axis_index("c")
    sid = lax.axis_index("s")
    wid = cid * 16 + sid
    NR = N * R
    per_tile = NR // 16
    seg = per_tile // 10
    epw = E // NW
    nch = epw // CH

    def inv_seg(k, _):
        off = sid * per_tile + k * seg
        sl = pl.ds(off, seg)
        pltpu.sync_copy(cnt_hbm.at[sl], c0)
        pltpu.sync_copy(cnt_hbm.at[pl.ds(NR + off, seg)], c1)

        def body(i, _):
            s16 = pl.ds(i * 16, 16)
            tot = c0[s16] + c1[s16]
            c0[s16] = 1.0 / jnp.maximum(tot, 1.0)
            return 0
        lax.fori_loop(0, seg // 16, body, 0)
        pltpu.sync_copy(c0, inv_sp.at[sl])
        return 0
    lax.fori_loop(0, 10, inv_seg, 0)
    plsc.subcore_barrier()

    pltpu.sync_copy(src_hbm.at[pl.ds(wid * epw, epw)], spre)
    pltpu.sync_copy(dst_hbm.at[pl.ds(wid * epw, epw)], dpre)
    pltpu.sync_copy(et_hbm.at[pl.ds(wid * epw, epw)], tpre)

    def keys(j, _):
        for g in range(CH // 16):
            fl = pl.ds(j * CH + g * 16, 16)
            kbuf2[j, pl.ds(g * 16, 16)] = dpre[fl] * R + tpre[fl]
            gflat[fl] = tpre[fl] + spre[fl] * R
            dflat[fl] = dpre[fl]
        return 0
    lax.fori_loop(0, nch, keys, 0)
    for i in range((EPT - 10000) // 16):
        fl = pl.ds(10000 + i * 16, 16)
        gflat[fl] = jnp.zeros((16,), jnp.int32)
        dflat[fl] = jnp.zeros((16,), jnp.int32)
        scflat[fl] = jnp.zeros((16,), jnp.float32)

    def run(sem):
        def fire(j, _):
            pltpu.async_copy(inv_sp.at[kbuf2.at[j]],
                             scflat.at[pl.ds(j * CH, CH)], sem)
            return 0
        lax.fori_loop(0, nch, fire, 0)
        pltpu.make_async_copy(scale_hbm.at[pl.ds(0, epw)],
                              scflat.at[pl.ds(0, epw)], sem).wait()
    pl.run_scoped(run, pltpu.SemaphoreType.DMA)

    base = wid * EPT
    pltpu.sync_copy(scflat, scale_hbm.at[pl.ds(base, EPT)])
    pltpu.sync_copy(gflat, gkey_hbm.at[pl.ds(base, EPT)])
    pltpu.sync_copy(dflat, dstp_hbm.at[pl.ds(base, EPT)])


def _make_scale(N, E):
    NR = N * R
    seg = NR // 160
    epw = E // NW
    E2 = NW * EPT
    return pl.kernel(
        functools.partial(_scale_body, N, E),
        out_type=(jax.ShapeDtypeStruct((E2,), jnp.float32),
                  jax.ShapeDtypeStruct((E2,), jnp.int32),
                  jax.ShapeDtypeStruct((E2,), jnp.int32)),
        mesh=_mesh(),
        scratch_types=[
            pltpu.VMEM((seg,), jnp.float32),
            pltpu.VMEM((seg,), jnp.float32),
            pltpu.VMEM((epw,), jnp.int32),
            pltpu.VMEM((epw,), jnp.int32),
            pltpu.VMEM((epw,), jnp.int32),
            pltpu.VMEM((epw // CH, CH), jnp.int32),
            pltpu.VMEM((EPT,), jnp.int32),
            pltpu.VMEM((EPT,), jnp.int32),
            pltpu.VMEM((EPT,), jnp.float32),
            pltpu.VMEM_SHARED((NR,), jnp.float32),
        ],
    )


PASSES = 5


def _agg_body(N, D, zt_hbm, gkey_hbm, dstp_hbm, scale_hbm, part_hbm,
              gk2, dst2, sc2, rowsA, rowsB, acc_sp):
    cid = lax.axis_index("c")
    sid = lax.axis_index("s")
    wid = cid * 16 + sid
    nch = EPT // CHM
    cpp = nch // PASSES
    seg_rows = 80
    row_base = sid * 640
    nseg = jnp.where(sid < 15, 8, (N - 15 * 640) // seg_rows)

    def zb(i, _):
        for f in range(D // 16):
            rowsA[i, pl.ds(f * 16, 16)] = jnp.zeros((16,), jnp.float32)
        return 0
    lax.fori_loop(0, seg_rows, zb, 0)

    def zcopy(k, _):
        pltpu.sync_copy(rowsA.at[pl.ds(0, seg_rows)],
                        acc_sp.at[pl.ds(row_base + k * seg_rows, seg_rows)])
        return 0
    lax.fori_loop(0, nseg, zcopy, 0)
    plsc.subcore_barrier()

    def scale_rows(rows, j):
        def grp(g, _):
            svec = sc2[j, pl.ds(g * 16, 16)]
            for i16 in range(16):
                s = jnp.take_along_axis(
                    svec, jnp.full((16,), i16, jnp.int32), axis=0)
                row = g * 16 + i16
                for f in range(D // 16):
                    sl = pl.ds(f * 16, 16)
                    rows[row, sl] = rows[row, sl] * s
            return 0
        lax.fori_loop(0, CHM // 16, grp, 0)

    def run(gsA, gsB, ssA, ssB):
        def gather(rows, j, sem):
            pltpu.async_copy(zt_hbm.at[gk2.at[j]], rows, sem)

        def gwait(rows, j, sem):
            pltpu.make_async_copy(zt_hbm.at[gk2.at[j]], rows, sem).wait()

        def scat(rows, j, sem):
            pltpu.async_copy(rows, acc_sp.at[dst2.at[j]], sem, add=True)

        def swait(rows, j, sem):
            pltpu.make_async_copy(rows, acc_sp.at[dst2.at[j]], sem).wait()

        for p in range(PASSES):
            rb = wid * nch + p * cpp
            pltpu.sync_copy(gkey_hbm.at[pl.ds(rb, cpp)], gk2)
            pltpu.sync_copy(dstp_hbm.at[pl.ds(rb, cpp)], dst2)
            pltpu.sync_copy(scale_hbm.at[pl.ds(rb, cpp)], sc2)
            gather(rowsA, 0, gsA)

            def pair(jj, _):
                j0 = jj * 2
                j1 = j0 + 1

                @pl.when(jj > 0)
                def _():
                    swait(rowsB, j0 - 1, ssB)
                gather(rowsB, j1, gsB)
                gwait(rowsA, j0, gsA)
                scale_rows(rowsA, j0)
                scat(rowsA, j0, ssA)
                gwait(rowsB, j1, gsB)
                scale_rows(rowsB, j1)
                scat(rowsB, j1, ssB)
                swait(rowsA, j0, ssA)

                @pl.when(jj < cpp // 2 - 1)
                def _():
                    gather(rowsA, j0 + 2, gsA)
                return 0
            lax.fori_loop(0, cpp // 2, pair, 0)
            swait(rowsB, cpp - 1, ssB)

    pl.run_scoped(run, pltpu.SemaphoreType.DMA, pltpu.SemaphoreType.DMA,
                  pltpu.SemaphoreType.DMA, pltpu.SemaphoreType.DMA)
    plsc.subcore_barrier()

    def ecopy(k, _):
        sl = pl.ds(row_base + k * seg_rows, seg_rows)
        pltpu.sync_copy(acc_sp.at[sl], rowsA.at[pl.ds(0, seg_rows)])
        pltpu.sync_copy(rowsA.at[pl.ds(0, seg_rows)], part_hbm.at[cid, sl])
        return 0
    lax.fori_loop(0, nseg, ecopy, 0)


def _make_agg(N, D):
    nch = EPT // CHM
    cpp = nch // PASSES
    return pl.kernel(
        functools.partial(_agg_body, N, D),
        out_type=jax.ShapeDtypeStruct((2, N, D), jnp.float32),
        mesh=_mesh(),
        scratch_types=[
            pltpu.VMEM((cpp, CHM), jnp.int32),
            pltpu.VMEM((cpp, CHM), jnp.int32),
            pltpu.VMEM((cpp, CHM), jnp.float32),
            pltpu.VMEM((CHM, D), jnp.float32),
            pltpu.VMEM((CHM, D), jnp.float32),
            pltpu.VMEM_SHARED((N, D), jnp.float32),
        ],
    )


def _wbuild(comp_ref, bases_ref, root_ref, w_ref):
    B, I, O = bases_ref.shape
    bflat = bases_ref[...].reshape(B, I * O)
    w = jnp.dot(comp_ref[...], bflat, preferred_element_type=jnp.float32)
    w_ref[0:R] = w.reshape(R, I, O)
    w_ref[R] = root_ref[...]


def _z1_kernel(f_ref, comp_ref, bases_ref, root_ref, orel_ref, oroot_ref, w_ref):
    @pl.when(pl.program_id(0) == 0)
    def _():
        _wbuild(comp_ref, bases_ref, root_ref, w_ref)
    f = f_ref[...]
    for r in range(R):
        orel_ref[:, r, :] = jnp.dot(f, w_ref[r], preferred_element_type=jnp.float32)
    oroot_ref[...] = jnp.dot(f, w_ref[R], preferred_element_type=jnp.float32)


def _z1_all(feats, comp, bases, root):
    N, I = feats.shape
    O = bases.shape[2]
    B = bases.shape[0]
    nb = N // Z_BLK
    return pl.pallas_call(
        _z1_kernel,
        grid=(nb,),
        in_specs=[
            pl.BlockSpec((Z_BLK, I), lambda n: (n, 0)),
            pl.BlockSpec((R, B), lambda n: (0, 0)),
            pl.BlockSpec((B, I, O), lambda n: (0, 0, 0)),
            pl.BlockSpec((I, O), lambda n: (0, 0)),
        ],
        out_specs=[
            pl.BlockSpec((Z_BLK, R, O), lambda n: (n, 0, 0)),
            pl.BlockSpec((Z_BLK, O), lambda n: (n, 0)),
        ],
        out_shape=[
            jax.ShapeDtypeStruct((N, R, O), jnp.float32),
            jax.ShapeDtypeStruct((N, O), jnp.float32),
        ],
        scratch_shapes=[pltpu.VMEM((RP, I, O), jnp.float32)],
    )(feats, comp, bases, root)


def _z2_kernel(part_ref, zroot_ref, b_ref, comp_ref, bases_ref, root_ref,
               orel_ref, oroot_ref, w_ref):
    @pl.when(pl.program_id(0) == 0)
    def _():
        _wbuild(comp_ref, bases_ref, root_ref, w_ref)
    s = part_ref[0] + part_ref[1] + zroot_ref[...] + b_ref[...]
    f = jnp.maximum(s, 0.0)
    for r in range(R):
        orel_ref[:, r, :] = jnp.dot(f, w_ref[r], preferred_element_type=jnp.float32)
    oroot_ref[...] = jnp.dot(f, w_ref[R], preferred_element_type=jnp.float32)


def _z2_all(part, zroot, bias, comp, bases, root):
    _, N, O = part.shape
    I = O
    B = bases.shape[0]
    nb = N // Z_BLK
    return pl.pallas_call(
        _z2_kernel,
        grid=(nb,),
        in_specs=[
            pl.BlockSpec((2, Z_BLK, O), lambda n: (0, n, 0)),
            pl.BlockSpec((Z_BLK, O), lambda n: (n, 0)),
            pl.BlockSpec((1, O), lambda n: (0, 0)),
            pl.BlockSpec((R, B), lambda n: (0, 0)),
            pl.BlockSpec((B, I, O), lambda n: (0, 0, 0)),
            pl.BlockSpec((I, O), lambda n: (0, 0)),
        ],
        out_specs=[
            pl.BlockSpec((Z_BLK, R, O), lambda n: (n, 0, 0)),
            pl.BlockSpec((Z_BLK, O), lambda n: (n, 0)),
        ],
        out_shape=[
            jax.ShapeDtypeStruct((N, R, O), jnp.float32),
            jax.ShapeDtypeStruct((N, O), jnp.float32),
        ],
        scratch_shapes=[pltpu.VMEM((RP, I, O), jnp.float32)],
    )(part, zroot, bias.reshape(1, O), comp, bases, root)


def _final_kernel(part_ref, z_ref, b_ref, wc_ref, bc_ref, out_ref):
    h = part_ref[0] + part_ref[1] + z_ref[...] + b_ref[...]
    out_ref[...] = jnp.dot(h, wc_ref[...], preferred_element_type=jnp.float32) + bc_ref[...]


def _final(part, zroot, bias, Wc, bc):
    _, N, O = part.shape
    A = Wc.shape[1]
    nb = N // NB_BLK
    return pl.pallas_call(
        _final_kernel,
        grid=(nb,),
        in_specs=[
            pl.BlockSpec((2, NB_BLK, O), lambda n: (0, n, 0)),
            pl.BlockSpec((NB_BLK, O), lambda n: (n, 0)),
            pl.BlockSpec((1, O), lambda n: (0, 0)),
            pl.BlockSpec((O, A), lambda n: (0, 0)),
            pl.BlockSpec((1, A), lambda n: (0, 0)),
        ],
        out_specs=pl.BlockSpec((NB_BLK, A), lambda n: (n, 0)),
        out_shape=jax.ShapeDtypeStruct((N, A), jnp.float32),
    )(part, zroot, bias.reshape(1, O), Wc, bc.reshape(1, A))


@jax.jit
def kernel(x, edge_index, edge_type, bases1, comp1, root1, bias1,
           bases2, comp2, root2, bias2, Wc, bc):
    N, I = x.shape
    E = edge_index.shape[1]
    nch = EPT // CHM
    src = edge_index[0]
    dst = edge_index[1]
    et = edge_type

    cnt = _make_hist(N, E)(dst, et)
    scale, gkey, dstp = _make_scale(N, E)(src, dst, et, cnt)
    gkey2 = gkey.reshape(NW * nch, CHM)
    dstp2 = dstp.reshape(NW * nch, CHM)
    scale2 = scale.reshape(NW * nch, CHM)

    agg = _make_agg(N, I)

    z1, z1root = _z1_all(x, comp1, bases1, root1)
    part1 = agg(z1.reshape(N * R, I), gkey2, dstp2, scale2)
    z2, z2root = _z2_all(part1, z1root, bias1, comp2, bases2, root2)
    part2 = agg(z2.reshape(N * R, I), gkey2, dstp2, scale2)
    return _final(part2, z2root, bias2, Wc, bc)
